# Initial kernel scaffold; baseline (speedup 1.0000x reference)
import jax
import jax.numpy as jnp
from jax.experimental import pallas as pl


def _bcast_body(tab_ref, out_ref):
    out_ref[...] = jnp.broadcast_to(tab_ref[None, :, :], out_ref.shape)


def kernel(x, pos_embedding):
    b = x.shape[0]
    s, h = pos_embedding.shape
    block_s = 512
    out = pl.pallas_call(
        _bcast_body,
        grid=(s // block_s,),
        in_specs=[pl.BlockSpec((block_s, h), lambda i: (i, 0))],
        out_specs=pl.BlockSpec((b, block_s, h), lambda i: (0, i, 0)),
        out_shape=jax.ShapeDtypeStruct((b, s, h), pos_embedding.dtype),
    )(pos_embedding)
    return out


# speedup vs baseline: 5.0439x; 5.0439x over previous
import jax
import jax.numpy as jnp
from jax.experimental import pallas as pl


def _bcast_body(tab_ref, out_ref):
    block = tab_ref[...]
    out_ref[...] = jnp.broadcast_to(block[None, :, :], out_ref.shape)


def kernel(x, pos_embedding):
    b = x.shape[0]
    s, h = pos_embedding.shape
    block_s = 512
    out = pl.pallas_call(
        _bcast_body,
        grid=(s // block_s,),
        in_specs=[pl.BlockSpec((block_s, h), lambda i: (i, 0))],
        out_specs=pl.BlockSpec((b, block_s, h), lambda i: (0, i, 0)),
        out_shape=jax.ShapeDtypeStruct((b, s, h), pos_embedding.dtype),
    )(pos_embedding)
    return out
